# Initial kernel scaffold; baseline (speedup 1.0000x reference)
import jax
import jax.numpy as jnp
from jax.experimental import pallas as pl


def kernel(context_train, query_train, Wcf, Ucf, bcf, Wcb, Ucb, bcb, Wqf, Uqf, bqf, Wqb, Uqb, bqb):
    raise NotImplementedError("write your pallas kernel here")


# speedup vs baseline: 6.0560x; 6.0560x over previous
import functools

import jax
import jax.numpy as jnp
from jax.experimental import pallas as pl
from jax.experimental.pallas import tpu as pltpu

N = 100
NP = 128
H2 = 2 * NP
G4 = 4 * H2


def _lstm_pair_kernel(x_ref, w_ref, u_ref, b_ref, hs_ref, h_ref, c_ref, xw_ref):
    @pl.when(pl.program_id(0) == 0)
    def _init():
        h_ref[...] = jnp.zeros_like(h_ref)
        c_ref[...] = jnp.zeros_like(c_ref)

    xw_ref[...] = (
        jnp.dot(x_ref[...], w_ref[...], preferred_element_type=jnp.float32)
        + b_ref[...]
    )

    chunk = x_ref.shape[0]

    def step(t, carry):
        h, c = carry
        z = xw_ref[pl.ds(t, 1), :] + jnp.dot(
            h, u_ref[...], preferred_element_type=jnp.float32
        )
        i = jax.nn.sigmoid(z[:, 0:H2])
        f = jax.nn.sigmoid(z[:, H2:2 * H2])
        g = jnp.tanh(z[:, 2 * H2:3 * H2])
        o = jax.nn.sigmoid(z[:, 3 * H2:4 * H2])
        c_new = f * c + i * g
        h_new = o * jnp.tanh(c_new)
        hs_ref[pl.ds(t, 1), :] = h_new
        return h_new, c_new

    h, c = jax.lax.fori_loop(0, chunk, step, (h_ref[...], c_ref[...]))
    h_ref[...] = h
    c_ref[...] = c


def _run_pair(x_all, w_big, u_big, b_big, chunk):
    t_total = x_all.shape[0]
    grid = (t_total // chunk,)
    return pl.pallas_call(
        _lstm_pair_kernel,
        grid=grid,
        in_specs=[
            pl.BlockSpec((chunk, H2), lambda i: (i, 0)),
            pl.BlockSpec((H2, G4), lambda i: (0, 0)),
            pl.BlockSpec((H2, G4), lambda i: (0, 0)),
            pl.BlockSpec((1, G4), lambda i: (0, 0)),
        ],
        out_specs=pl.BlockSpec((chunk, H2), lambda i: (i, 0)),
        out_shape=jax.ShapeDtypeStruct((t_total, H2), jnp.float32),
        scratch_shapes=[
            pltpu.VMEM((1, H2), jnp.float32),
            pltpu.VMEM((1, H2), jnp.float32),
            pltpu.VMEM((chunk, G4), jnp.float32),
        ],
    )(x_all, w_big, u_big, b_big)


def _pack_pair(Mf, Mb):
    big = jnp.zeros((H2, G4), dtype=jnp.float32)
    for g in range(4):
        big = big.at[0:N, g * H2 + 0:g * H2 + N].set(Mf[:, g * N:(g + 1) * N])
        big = big.at[NP:NP + N, g * H2 + NP:g * H2 + NP + N].set(
            Mb[:, g * N:(g + 1) * N])
    return big


def _pack_bias(bf, bb):
    big = jnp.zeros((1, G4), dtype=jnp.float32)
    for g in range(4):
        big = big.at[0, g * H2 + 0:g * H2 + N].set(bf[g * N:(g + 1) * N])
        big = big.at[0, g * H2 + NP:g * H2 + NP + N].set(bb[g * N:(g + 1) * N])
    return big


def _pack_inputs(x, t_pad):
    t_real = x.shape[0]
    x_all = jnp.zeros((t_pad, H2), dtype=jnp.float32)
    x_all = x_all.at[:t_real, 0:N].set(x)
    x_all = x_all.at[:t_real, NP:NP + N].set(x[::-1])
    return x_all


def _bilstm_pallas(x, Wf, Uf, bf, Wb, Ub, bb, chunk):
    t_real = x.shape[0]
    t_pad = ((t_real + chunk - 1) // chunk) * chunk
    x_all = _pack_inputs(x, t_pad)
    w_big = _pack_pair(Wf, Wb)
    u_big = _pack_pair(Uf, Ub)
    b_big = _pack_bias(bf, bb)
    hs = _run_pair(x_all, w_big, u_big, b_big, chunk)
    fwd = hs[:t_real, 0:N]
    bwd = hs[:t_real, NP:NP + N][::-1]
    return jnp.concatenate([fwd, bwd], axis=-1)


@jax.jit
def kernel(context_train, query_train, Wcf, Ucf, bcf, Wcb, Ucb, bcb,
           Wqf, Uqf, bqf, Wqb, Uqb, bqb):
    h = _bilstm_pallas(context_train, Wcf, Ucf, bcf, Wcb, Ucb, bcb, chunk=512)
    u = _bilstm_pallas(query_train, Wqf, Uqf, bqf, Wqb, Uqb, bqb, chunk=200)
    return (h[None], u[None])


# speedup vs baseline: 8.2234x; 1.3579x over previous
import jax
import jax.numpy as jnp
from jax.experimental import pallas as pl
from jax.experimental.pallas import tpu as pltpu

N = 100
NP = 128
G4 = 4 * NP


def _lstm_pair_kernel(xf_ref, xb_ref, wf_ref, wb_ref, uf_ref, ub_ref,
                      bf_ref, bb_ref, hsf_ref, hsb_ref,
                      cf_ref, rf_ref, cb_ref, rb_ref, xwf_ref, xwb_ref):
    @pl.when(pl.program_id(0) == 0)
    def _init():
        cf_ref[...] = jnp.zeros_like(cf_ref)
        rf_ref[...] = jnp.zeros_like(rf_ref)
        cb_ref[...] = jnp.zeros_like(cb_ref)
        rb_ref[...] = jnp.zeros_like(rb_ref)

    xwf_ref[...] = (
        jnp.dot(xf_ref[...], wf_ref[...], preferred_element_type=jnp.float32)
        + bf_ref[...]
    )
    xwb_ref[...] = (
        jnp.dot(xb_ref[...], wb_ref[...], preferred_element_type=jnp.float32)
        + bb_ref[...]
    )

    chunk = xf_ref.shape[0]

    def one_dir(t, c, r, xw_ref, u_ref, out_ref):
        z = xw_ref[pl.ds(t, 1), :] + r
        i = jax.nn.sigmoid(z[:, 0:NP])
        f = jax.nn.sigmoid(z[:, NP:2 * NP])
        g = jnp.tanh(z[:, 2 * NP:3 * NP])
        o = jax.nn.sigmoid(z[:, 3 * NP:4 * NP])
        c_new = f * c + i * g
        h_new = o * jnp.tanh(c_new)
        out_ref[pl.ds(t, 1), :] = h_new
        r_new = jnp.dot(h_new, u_ref[...], preferred_element_type=jnp.float32)
        return c_new, r_new

    def step(t, carry):
        cf, rf, cb, rb = carry
        cf, rf = one_dir(t, cf, rf, xwf_ref, uf_ref, hsf_ref)
        cb, rb = one_dir(t, cb, rb, xwb_ref, ub_ref, hsb_ref)
        return cf, rf, cb, rb

    carry0 = (cf_ref[...], rf_ref[...], cb_ref[...], rb_ref[...])
    cf, rf, cb, rb = jax.lax.fori_loop(0, chunk, step, carry0, unroll=4)
    cf_ref[...] = cf
    rf_ref[...] = rf
    cb_ref[...] = cb
    rb_ref[...] = rb


def _run_pair(xf, xb, wf, wb, uf, ub, bf, bb, chunk):
    t_total = xf.shape[0]
    grid = (t_total // chunk,)
    wspec = pl.BlockSpec((NP, G4), lambda i: (0, 0))
    bspec = pl.BlockSpec((1, G4), lambda i: (0, 0))
    xspec = pl.BlockSpec((chunk, NP), lambda i: (i, 0))
    return pl.pallas_call(
        _lstm_pair_kernel,
        grid=grid,
        in_specs=[xspec, xspec, wspec, wspec, wspec, wspec, bspec, bspec],
        out_specs=[pl.BlockSpec((chunk, NP), lambda i: (i, 0)),
                   pl.BlockSpec((chunk, NP), lambda i: (i, 0))],
        out_shape=[jax.ShapeDtypeStruct((t_total, NP), jnp.float32),
                   jax.ShapeDtypeStruct((t_total, NP), jnp.float32)],
        scratch_shapes=[
            pltpu.VMEM((1, NP), jnp.float32),
            pltpu.VMEM((1, G4), jnp.float32),
            pltpu.VMEM((1, NP), jnp.float32),
            pltpu.VMEM((1, G4), jnp.float32),
            pltpu.VMEM((chunk, G4), jnp.float32),
            pltpu.VMEM((chunk, G4), jnp.float32),
        ],
    )(xf, xb, wf, wb, uf, ub, bf, bb)


def _pack_w(M):
    big = jnp.zeros((NP, G4), dtype=jnp.float32)
    for g in range(4):
        big = big.at[0:N, g * NP:g * NP + N].set(M[:, g * N:(g + 1) * N])
    return big


def _pack_b(b):
    big = jnp.zeros((1, G4), dtype=jnp.float32)
    for g in range(4):
        big = big.at[0, g * NP:g * NP + N].set(b[g * N:(g + 1) * N])
    return big


def _pad_x(x, t_pad):
    t_real = x.shape[0]
    xp = jnp.zeros((t_pad, NP), dtype=jnp.float32)
    return xp.at[:t_real, 0:N].set(x)


def _bilstm_pallas(x, Wf, Uf, bf, Wb, Ub, bb, chunk):
    t_real = x.shape[0]
    t_pad = ((t_real + chunk - 1) // chunk) * chunk
    xf = _pad_x(x, t_pad)
    xb = _pad_x(x[::-1], t_pad)
    hsf, hsb = _run_pair(xf, xb, _pack_w(Wf), _pack_w(Wb), _pack_w(Uf),
                         _pack_w(Ub), _pack_b(bf), _pack_b(bb), chunk)
    fwd = hsf[:t_real, 0:N]
    bwd = hsb[:t_real, 0:N][::-1]
    return jnp.concatenate([fwd, bwd], axis=-1)


@jax.jit
def kernel(context_train, query_train, Wcf, Ucf, bcf, Wcb, Ucb, bcb,
           Wqf, Uqf, bqf, Wqb, Uqb, bqb):
    h = _bilstm_pallas(context_train, Wcf, Ucf, bcf, Wcb, Ucb, bcb, chunk=512)
    u = _bilstm_pallas(query_train, Wqf, Uqf, bqf, Wqb, Uqb, bqb, chunk=200)
    return (h[None], u[None])


# speedup vs baseline: 8.3366x; 1.0138x over previous
import jax
import jax.numpy as jnp
from jax.experimental import pallas as pl
from jax.experimental.pallas import tpu as pltpu

N = 100
NP = 128
G4 = 4 * NP


def _lstm_pair_kernel(xf_ref, xb_ref, wf_ref, wb_ref, uf_ref, ub_ref,
                      bf_ref, bb_ref, hsf_ref, hsb_ref,
                      cf_ref, rf_ref, cb_ref, rb_ref, xwf_ref, xwb_ref):
    @pl.when(pl.program_id(0) == 0)
    def _init():
        cf_ref[...] = jnp.zeros_like(cf_ref)
        rf_ref[...] = jnp.zeros_like(rf_ref)
        cb_ref[...] = jnp.zeros_like(cb_ref)
        rb_ref[...] = jnp.zeros_like(rb_ref)

    xwf_ref[...] = (
        jnp.dot(xf_ref[...], wf_ref[...], preferred_element_type=jnp.float32)
        + bf_ref[...]
    )
    xwb_ref[...] = (
        jnp.dot(xb_ref[...], wb_ref[...], preferred_element_type=jnp.float32)
        + bb_ref[...]
    )

    chunk = xf_ref.shape[0]

    def one_dir(t, c, r, xw_ref, u_ref, out_ref):
        z = xw_ref[pl.ds(t, 1), :] + r
        i = jax.nn.sigmoid(z[:, 0:NP])
        f = jax.nn.sigmoid(z[:, NP:2 * NP])
        g = jnp.tanh(z[:, 2 * NP:3 * NP])
        o = jax.nn.sigmoid(z[:, 3 * NP:4 * NP])
        c_new = f * c + i * g
        h_new = o * jnp.tanh(c_new)
        out_ref[pl.ds(t, 1), :] = h_new
        r_new = jnp.dot(h_new, u_ref[...], preferred_element_type=jnp.float32)
        return c_new, r_new

    def step(t, carry):
        cf, rf, cb, rb = carry
        cf, rf = one_dir(t, cf, rf, xwf_ref, uf_ref, hsf_ref)
        cb, rb = one_dir(t, cb, rb, xwb_ref, ub_ref, hsb_ref)
        return cf, rf, cb, rb

    carry0 = (cf_ref[...], rf_ref[...], cb_ref[...], rb_ref[...])
    cf, rf, cb, rb = jax.lax.fori_loop(0, chunk, step, carry0, unroll=8)
    cf_ref[...] = cf
    rf_ref[...] = rf
    cb_ref[...] = cb
    rb_ref[...] = rb


def _run_pair(xf, xb, wf, wb, uf, ub, bf, bb, chunk):
    t_total = xf.shape[0]
    grid = (t_total // chunk,)
    wspec = pl.BlockSpec((NP, G4), lambda i: (0, 0))
    bspec = pl.BlockSpec((1, G4), lambda i: (0, 0))
    xspec = pl.BlockSpec((chunk, NP), lambda i: (i, 0))
    return pl.pallas_call(
        _lstm_pair_kernel,
        grid=grid,
        in_specs=[xspec, xspec, wspec, wspec, wspec, wspec, bspec, bspec],
        out_specs=[pl.BlockSpec((chunk, NP), lambda i: (i, 0)),
                   pl.BlockSpec((chunk, NP), lambda i: (i, 0))],
        out_shape=[jax.ShapeDtypeStruct((t_total, NP), jnp.float32),
                   jax.ShapeDtypeStruct((t_total, NP), jnp.float32)],
        scratch_shapes=[
            pltpu.VMEM((1, NP), jnp.float32),
            pltpu.VMEM((1, G4), jnp.float32),
            pltpu.VMEM((1, NP), jnp.float32),
            pltpu.VMEM((1, G4), jnp.float32),
            pltpu.VMEM((chunk, G4), jnp.float32),
            pltpu.VMEM((chunk, G4), jnp.float32),
        ],
    )(xf, xb, wf, wb, uf, ub, bf, bb)


def _pack_w(M):
    big = jnp.zeros((NP, G4), dtype=jnp.float32)
    for g in range(4):
        big = big.at[0:N, g * NP:g * NP + N].set(M[:, g * N:(g + 1) * N])
    return big


def _pack_b(b):
    big = jnp.zeros((1, G4), dtype=jnp.float32)
    for g in range(4):
        big = big.at[0, g * NP:g * NP + N].set(b[g * N:(g + 1) * N])
    return big


def _pad_x(x, t_pad):
    t_real = x.shape[0]
    xp = jnp.zeros((t_pad, NP), dtype=jnp.float32)
    return xp.at[:t_real, 0:N].set(x)


def _bilstm_pallas(x, Wf, Uf, bf, Wb, Ub, bb, chunk):
    t_real = x.shape[0]
    t_pad = ((t_real + chunk - 1) // chunk) * chunk
    xf = _pad_x(x, t_pad)
    xb = _pad_x(x[::-1], t_pad)
    hsf, hsb = _run_pair(xf, xb, _pack_w(Wf), _pack_w(Wb), _pack_w(Uf),
                         _pack_w(Ub), _pack_b(bf), _pack_b(bb), chunk)
    fwd = hsf[:t_real, 0:N]
    bwd = hsb[:t_real, 0:N][::-1]
    return jnp.concatenate([fwd, bwd], axis=-1)


@jax.jit
def kernel(context_train, query_train, Wcf, Ucf, bcf, Wcb, Ucb, bcb,
           Wqf, Uqf, bqf, Wqb, Uqb, bqb):
    h = _bilstm_pallas(context_train, Wcf, Ucf, bcf, Wcb, Ucb, bcb, chunk=512)
    u = _bilstm_pallas(query_train, Wqf, Uqf, bqf, Wqb, Uqb, bqb, chunk=200)
    return (h[None], u[None])


# speedup vs baseline: 8.3433x; 1.0008x over previous
import jax
import jax.numpy as jnp
from jax.experimental import pallas as pl
from jax.experimental.pallas import tpu as pltpu

N = 100
NP = 128
G4 = 4 * NP


def _lstm_pair_kernel(xf_ref, xb_ref, wf_ref, wb_ref, uf_ref, ub_ref,
                      bf_ref, bb_ref, hsf_ref, hsb_ref,
                      cf_ref, rf_ref, cb_ref, rb_ref, xwf_ref, xwb_ref):
    @pl.when(pl.program_id(0) == 0)
    def _init():
        cf_ref[...] = jnp.zeros_like(cf_ref)
        rf_ref[...] = jnp.zeros_like(rf_ref)
        cb_ref[...] = jnp.zeros_like(cb_ref)
        rb_ref[...] = jnp.zeros_like(rb_ref)

    xwf_ref[...] = (
        jnp.dot(xf_ref[...], wf_ref[...], preferred_element_type=jnp.float32)
        + bf_ref[...]
    )
    xwb_ref[...] = (
        jnp.dot(xb_ref[...], wb_ref[...], preferred_element_type=jnp.float32)
        + bb_ref[...]
    )

    chunk = xf_ref.shape[0]

    def one_dir(t, c, r, xw_ref, u_ref, out_ref):
        z = xw_ref[pl.ds(t, 1), :] + r
        i = jax.nn.sigmoid(z[:, 0:NP])
        f = jax.nn.sigmoid(z[:, NP:2 * NP])
        g = jnp.tanh(z[:, 2 * NP:3 * NP])
        o = jax.nn.sigmoid(z[:, 3 * NP:4 * NP])
        c_new = f * c + i * g
        h_new = o * jnp.tanh(c_new)
        out_ref[pl.ds(t, 1), :] = h_new
        r_new = jnp.dot(h_new.astype(jnp.bfloat16), u_ref[...],
                        preferred_element_type=jnp.float32)
        return c_new, r_new

    def step(t, carry):
        cf, rf, cb, rb = carry
        cf, rf = one_dir(t, cf, rf, xwf_ref, uf_ref, hsf_ref)
        cb, rb = one_dir(t, cb, rb, xwb_ref, ub_ref, hsb_ref)
        return cf, rf, cb, rb

    carry0 = (cf_ref[...], rf_ref[...], cb_ref[...], rb_ref[...])
    cf, rf, cb, rb = jax.lax.fori_loop(0, chunk, step, carry0, unroll=8)
    cf_ref[...] = cf
    rf_ref[...] = rf
    cb_ref[...] = cb
    rb_ref[...] = rb


def _run_pair(xf, xb, wf, wb, uf, ub, bf, bb, chunk):
    t_total = xf.shape[0]
    grid = (t_total // chunk,)
    uf = uf.astype(jnp.bfloat16)
    ub = ub.astype(jnp.bfloat16)
    wspec = pl.BlockSpec((NP, G4), lambda i: (0, 0))
    bspec = pl.BlockSpec((1, G4), lambda i: (0, 0))
    xspec = pl.BlockSpec((chunk, NP), lambda i: (i, 0))
    return pl.pallas_call(
        _lstm_pair_kernel,
        grid=grid,
        in_specs=[xspec, xspec, wspec, wspec, wspec, wspec, bspec, bspec],
        out_specs=[pl.BlockSpec((chunk, NP), lambda i: (i, 0)),
                   pl.BlockSpec((chunk, NP), lambda i: (i, 0))],
        out_shape=[jax.ShapeDtypeStruct((t_total, NP), jnp.float32),
                   jax.ShapeDtypeStruct((t_total, NP), jnp.float32)],
        scratch_shapes=[
            pltpu.VMEM((1, NP), jnp.float32),
            pltpu.VMEM((1, G4), jnp.float32),
            pltpu.VMEM((1, NP), jnp.float32),
            pltpu.VMEM((1, G4), jnp.float32),
            pltpu.VMEM((chunk, G4), jnp.float32),
            pltpu.VMEM((chunk, G4), jnp.float32),
        ],
    )(xf, xb, wf, wb, uf, ub, bf, bb)


def _pack_w(M):
    big = jnp.zeros((NP, G4), dtype=jnp.float32)
    for g in range(4):
        big = big.at[0:N, g * NP:g * NP + N].set(M[:, g * N:(g + 1) * N])
    return big


def _pack_b(b):
    big = jnp.zeros((1, G4), dtype=jnp.float32)
    for g in range(4):
        big = big.at[0, g * NP:g * NP + N].set(b[g * N:(g + 1) * N])
    return big


def _pad_x(x, t_pad):
    t_real = x.shape[0]
    xp = jnp.zeros((t_pad, NP), dtype=jnp.float32)
    return xp.at[:t_real, 0:N].set(x)


def _bilstm_pallas(x, Wf, Uf, bf, Wb, Ub, bb, chunk):
    t_real = x.shape[0]
    t_pad = ((t_real + chunk - 1) // chunk) * chunk
    xf = _pad_x(x, t_pad)
    xb = _pad_x(x[::-1], t_pad)
    hsf, hsb = _run_pair(xf, xb, _pack_w(Wf), _pack_w(Wb), _pack_w(Uf),
                         _pack_w(Ub), _pack_b(bf), _pack_b(bb), chunk)
    fwd = hsf[:t_real, 0:N]
    bwd = hsb[:t_real, 0:N][::-1]
    return jnp.concatenate([fwd, bwd], axis=-1)


@jax.jit
def kernel(context_train, query_train, Wcf, Ucf, bcf, Wcb, Ucb, bcb,
           Wqf, Uqf, bqf, Wqb, Uqb, bqb):
    h = _bilstm_pallas(context_train, Wcf, Ucf, bcf, Wcb, Ucb, bcb, chunk=512)
    u = _bilstm_pallas(query_train, Wqf, Uqf, bqf, Wqb, Uqb, bqb, chunk=200)
    return (h[None], u[None])
